# Initial kernel scaffold; baseline (speedup 1.0000x reference)
import jax
import jax.numpy as jnp
from jax.experimental import pallas as pl


def kernel(mem, feature, rel_logits, slot_idx):
    raise NotImplementedError("write your pallas kernel here")


# speedup vs baseline: 19.0605x; 19.0605x over previous
import functools

import jax
import jax.numpy as jnp
from jax import lax
from jax.experimental import pallas as pl
from jax.experimental.pallas import tpu as pltpu
from jax.experimental.pallas import tpu_sc as plsc

_MAX_SIZE = 4096
_N_PROTO = 51
_FEAT_DIM = 128
_N_REL = 16384
_INV_THRESH = 1.0 / 0.9
_LN9 = 2.1972245773362196

_NC = 2
_NS = 16
_NW = _NC * _NS
_CHUNK = _N_REL // _NW
_L = 16
_NG = _CHUNK // _L


def _compute_body(logits_hbm, slot_hbm, targ_hbm, cnt_hbm,
                  logits_v, slot_v, targ_v, acc_v, cnt_v):
  wid = lax.axis_index("s") * _NC + lax.axis_index("c")
  base = wid * _CHUNK
  pltpu.sync_copy(slot_hbm.at[pl.ds(base, _CHUNK)], slot_v)
  pltpu.sync_copy(logits_hbm.at[:, pl.ds(base, _CHUNK)], logits_v)

  def group(g, acc):
    off = g * _L
    sl = pl.ds(off, _L)
    m = logits_v[0, sl]
    m2 = jnp.full((_L,), -jnp.inf, jnp.float32)
    amax = jnp.zeros((_L,), jnp.int32)
    for c in range(1, _N_PROTO):
      z = logits_v[c, sl]
      gt = z > m
      amax = jnp.where(gt, c, amax)
      m2 = jnp.maximum(m2, jnp.minimum(z, m))
      m = jnp.maximum(m, z)
    maybe = jnp.where(m - m2 > _LN9, 1.0, 0.0)
    mbv = maybe[0]
    for i in range(1, _L):
      mbv = mbv + maybe[i]

    targ = amax * _MAX_SIZE + slot_v[sl]
    targ_v[sl] = jnp.full((_L,), -1, jnp.int32)

    @pl.when(mbv > 0.0)
    def _exact():
      ssum = jnp.zeros((_L,), jnp.float32)
      for c in range(_N_PROTO):
        ssum = ssum + jnp.exp(logits_v[c, sl] - m)
      selv = jnp.where(ssum < _INV_THRESH, 1.0, 0.0)
      targ_v[sl] = jnp.where(ssum < _INV_THRESH, targ, -1)
      acc_v[...] = acc_v[...] + selv

    return acc

  acc_v[...] = jnp.zeros((_L,), jnp.float32)
  lax.fori_loop(0, _NG, group, 0)
  pltpu.sync_copy(targ_v, targ_hbm.at[pl.ds(base, _CHUNK)])
  cnt_v[...] = acc_v[...]
  pltpu.sync_copy(cnt_v, cnt_hbm.at[wid])


def _scatter_body(feature_hbm, targ_hbm, cnt_hbm, mem_ref,
                  targ_v, cnt_v, row_v):
  wid = lax.axis_index("s") * _NC + lax.axis_index("c")
  base = wid * _CHUNK
  pltpu.sync_copy(cnt_hbm.at[wid], cnt_v)
  avals = cnt_v[...]
  cnt = avals[0]
  for i in range(1, _L):
    cnt = cnt + avals[i]

  @pl.when(cnt > 0.0)
  def _scatter_rare():
    pltpu.sync_copy(targ_hbm.at[pl.ds(base, _CHUNK)], targ_v)

    def wgroup(g, carry):
      off = g * _L
      targ = targ_v[pl.ds(off, _L)]
      for i in range(_L):
        @pl.when(targ[i] >= 0)
        def _write():
          pltpu.sync_copy(feature_hbm.at[pl.ds(base + off + i, 1), :], row_v)
          pltpu.sync_copy(row_v, mem_ref.at[pl.ds(targ[i], 1), :])
      return carry

    lax.fori_loop(0, _NG, wgroup, 0)


_mesh = plsc.VectorSubcoreMesh(core_axis_name="c", subcore_axis_name="s")

_compute = pl.kernel(
    _compute_body,
    out_type=(
        jax.ShapeDtypeStruct((_N_REL,), jnp.int32),
        jax.ShapeDtypeStruct((_NW, _L), jnp.float32),
    ),
    mesh=_mesh,
    scratch_types=[
        pltpu.VMEM((_N_PROTO, _CHUNK), jnp.float32),
        pltpu.VMEM((_CHUNK,), jnp.int32),
        pltpu.VMEM((_CHUNK,), jnp.int32),
        pltpu.VMEM((_L,), jnp.float32),
        pltpu.VMEM((_L,), jnp.float32),
    ],
    name="memory_banks_route",
)

_scatter = pl.kernel(
    _scatter_body,
    out_type=(),
    mesh=_mesh,
    scratch_types=[
        pltpu.VMEM((_CHUNK,), jnp.int32),
        pltpu.VMEM((_L,), jnp.float32),
        pltpu.VMEM((1, _FEAT_DIM), jnp.float32),
    ],
    name="memory_banks_scatter",
)


def kernel(mem, feature, rel_logits, slot_idx):
  logits_t = rel_logits.T
  targ_enc, cnts = _compute(logits_t, slot_idx)
  mem_ref = jax.new_ref(mem)
  _scatter(feature, targ_enc, cnts, mem_ref)
  return mem_ref[...]
